# baseline (speedup 1.0000x reference)
import functools

import jax
import jax.numpy as jnp
from jax import lax
from jax.experimental import pallas as pl
from jax.experimental.pallas import tpu as pltpu
from jax.experimental.pallas import tpu_sc as plsc

BATCH = 16384
EMBED = 32
NUM_CORES = 2
NUM_SUBCORES = 16
LANES = 16
NUM_WORKERS = NUM_CORES * NUM_SUBCORES
BPW = BATCH // NUM_WORKERS
NBUF = 4
AHEAD = NBUF - 1

_mesh = plsc.VectorSubcoreMesh(core_axis_name="c", subcore_axis_name="s")


@functools.partial(
    pl.kernel,
    out_type=jax.ShapeDtypeStruct((BATCH,), jnp.float32),
    mesh=_mesh,
    compiler_params=pltpu.CompilerParams(needs_layout_passes=False,
                                         use_tc_tiling_on_sc=True),
    scratch_types=[
        pltpu.VMEM((BPW + LANES,), jnp.int32),
        pltpu.VMEM((BPW + LANES,), jnp.int32),
        pltpu.VMEM((NBUF, EMBED, 128), jnp.float32),
        pltpu.VMEM((NBUF, EMBED, 128), jnp.float32),
        pltpu.VMEM((BPW,), jnp.float32),
    ] + [pltpu.SemaphoreType.DMA] * (2 * NBUF),
)
def _mf_dot_kernel(xu_hbm, xi_hbm, wt_hbm, ht_hbm, out_hbm,
                   uidx, vidx, wtile, htile, outv, *sems):
    semw = sems[:NBUF]
    semh = sems[NBUF:]
    wid = lax.axis_index("s") * NUM_CORES + lax.axis_index("c")
    base = wid * BPW
    lanes = lax.iota(jnp.int32, LANES)

    pltpu.sync_copy(xu_hbm.at[pl.ds(base, BPW)], uidx.at[pl.ds(0, BPW)])
    pltpu.sync_copy(xi_hbm.at[pl.ds(base, BPW)], vidx.at[pl.ds(0, BPW)])
    uidx[pl.ds(BPW, LANES)] = jnp.zeros((LANES,), jnp.int32)
    vidx[pl.ds(BPW, LANES)] = jnp.zeros((LANES,), jnp.int32)

    def issue(u, v, slot):
        cu = pl.multiple_of((u >> 7) << 7, 128)
        cv = pl.multiple_of((v >> 7) << 7, 128)
        for r in range(EMBED // 8):
            pltpu.async_copy(
                wt_hbm.at[pl.ds(r * 8, 8), pl.ds(cu, 128)],
                wtile.at[slot, pl.ds(r * 8, 8)], semw[slot])
            pltpu.async_copy(
                ht_hbm.at[pl.ds(r * 8, 8), pl.ds(cv, 128)],
                htile.at[slot, pl.ds(r * 8, 8)], semh[slot])

    def drain(slot):
        pltpu.make_async_copy(wt_hbm.at[:, pl.ds(0, 128)],
                              wtile.at[slot], semw[slot]).wait()
        pltpu.make_async_copy(ht_hbm.at[:, pl.ds(0, 128)],
                              htile.at[slot], semh[slot]).wait()

    uvec0 = uidx[pl.ds(0, LANES)]
    vvec0 = vidx[pl.ds(0, LANES)]
    for p in range(AHEAD):
        issue(uvec0[p], vvec0[p], p)

    def group_body(g, carry):
        uvec, vvec = carry
        acc = jnp.zeros((LANES,), jnp.float32)
        unext, vnext = uvec, vvec
        for j in range(LANES):
            if j == LANES - AHEAD:
                unext = uidx[pl.ds((g + 1) * LANES, LANES)]
                vnext = vidx[pl.ds((g + 1) * LANES, LANES)]
            p = j + AHEAD
            if p < LANES:
                nu, nv = uvec[p], vvec[p]
            else:
                nu, nv = unext[p - LANES], vnext[p - LANES]
            slot = j % NBUF
            issue(nu, nv, p % NBUF)
            drain(slot)
            lu = jnp.broadcast_to(uvec[j] & 127, (LANES,))
            lv = jnp.broadcast_to(vvec[j] & 127, (LANES,))
            w0 = plsc.load_gather(wtile.at[slot], [lanes, lu])
            w1 = plsc.load_gather(wtile.at[slot], [lanes + LANES, lu])
            h0 = plsc.load_gather(htile.at[slot], [lanes, lv])
            h1 = plsc.load_gather(htile.at[slot], [lanes + LANES, lv])
            s = jnp.sum(w0 * h0 + w1 * h1)
            acc = jnp.where(lanes == j, s, acc)
        outv[pl.ds(g * LANES, LANES)] = acc
        return unext, vnext

    lax.fori_loop(0, BPW // LANES, group_body, (uvec0, vvec0))
    for p in range(AHEAD):
        drain(p % NBUF)

    pltpu.sync_copy(outv, out_hbm.at[pl.ds(base, BPW)])


def kernel(x, W, H):
    return _mf_dot_kernel(x[:, 0], x[:, 1], W.T, H.T)


# speedup vs baseline: 1.0007x; 1.0007x over previous
import functools

import jax
import jax.numpy as jnp
from jax import lax
from jax.experimental import pallas as pl
from jax.experimental.pallas import tpu as pltpu
from jax.experimental.pallas import tpu_sc as plsc

BATCH = 16384
EMBED = 32
NUM_CORES = 2
NUM_SUBCORES = 16
LANES = 16
NUM_WORKERS = NUM_CORES * NUM_SUBCORES
BPW = BATCH // NUM_WORKERS
NBUF = 4
AHEAD = NBUF - 1

_mesh = plsc.VectorSubcoreMesh(core_axis_name="c", subcore_axis_name="s")


@functools.partial(
    pl.kernel,
    out_type=jax.ShapeDtypeStruct((BATCH,), jnp.float32),
    mesh=_mesh,
    compiler_params=pltpu.CompilerParams(needs_layout_passes=False,
                                         use_tc_tiling_on_sc=True),
    scratch_types=[
        pltpu.VMEM((BPW + LANES,), jnp.int32),
        pltpu.VMEM((BPW + LANES,), jnp.int32),
        pltpu.VMEM((NBUF, EMBED, 128), jnp.float32),
        pltpu.VMEM((NBUF, EMBED, 128), jnp.float32),
        pltpu.VMEM((BPW,), jnp.float32),
    ] + [pltpu.SemaphoreType.DMA] * (2 * NBUF),
)
def _mf_dot_kernel(xu_hbm, xi_hbm, wt_hbm, ht_hbm, out_hbm,
                   uidx, vidx, wtile, htile, outv, *sems):
    semw = sems[:NBUF]
    semh = sems[NBUF:]
    wid = lax.axis_index("s") * NUM_CORES + lax.axis_index("c")
    base = wid * BPW
    lanes = lax.iota(jnp.int32, LANES)

    pltpu.sync_copy(xu_hbm.at[pl.ds(base, BPW)], uidx.at[pl.ds(0, BPW)])
    pltpu.sync_copy(xi_hbm.at[pl.ds(base, BPW)], vidx.at[pl.ds(0, BPW)])
    uidx[pl.ds(BPW, LANES)] = jnp.zeros((LANES,), jnp.int32)
    vidx[pl.ds(BPW, LANES)] = jnp.zeros((LANES,), jnp.int32)

    def issue(u, v, slot):
        cu = pl.multiple_of((u >> 7) << 7, 128)
        cv = pl.multiple_of((v >> 7) << 7, 128)
        pltpu.async_copy(wt_hbm.at[:, pl.ds(cu, 128)], wtile.at[slot],
                         semw[slot])
        pltpu.async_copy(ht_hbm.at[:, pl.ds(cv, 128)], htile.at[slot],
                         semh[slot])

    def drain(slot):
        pltpu.make_async_copy(wt_hbm.at[:, pl.ds(0, 128)],
                              wtile.at[slot], semw[slot]).wait()
        pltpu.make_async_copy(ht_hbm.at[:, pl.ds(0, 128)],
                              htile.at[slot], semh[slot]).wait()

    uvec0 = uidx[pl.ds(0, LANES)]
    vvec0 = vidx[pl.ds(0, LANES)]
    for p in range(AHEAD):
        issue(uvec0[p], vvec0[p], p)

    def group_body(g, carry):
        uvec, vvec = carry
        acc = jnp.zeros((LANES,), jnp.float32)
        unext, vnext = uvec, vvec
        for j in range(LANES):
            if j == LANES - AHEAD:
                unext = uidx[pl.ds((g + 1) * LANES, LANES)]
                vnext = vidx[pl.ds((g + 1) * LANES, LANES)]
            p = j + AHEAD
            if p < LANES:
                nu, nv = uvec[p], vvec[p]
            else:
                nu, nv = unext[p - LANES], vnext[p - LANES]
            slot = j % NBUF
            issue(nu, nv, p % NBUF)
            drain(slot)
            lu = jnp.broadcast_to(uvec[j] & 127, (LANES,))
            lv = jnp.broadcast_to(vvec[j] & 127, (LANES,))
            w0 = plsc.load_gather(wtile.at[slot], [lanes, lu])
            w1 = plsc.load_gather(wtile.at[slot], [lanes + LANES, lu])
            h0 = plsc.load_gather(htile.at[slot], [lanes, lv])
            h1 = plsc.load_gather(htile.at[slot], [lanes + LANES, lv])
            s = jnp.sum(w0 * h0 + w1 * h1)
            acc = jnp.where(lanes == j, s, acc)
        outv[pl.ds(g * LANES, LANES)] = acc
        return unext, vnext

    lax.fori_loop(0, BPW // LANES, group_body, (uvec0, vvec0))
    for p in range(AHEAD):
        drain(p % NBUF)

    pltpu.sync_copy(outv, out_hbm.at[pl.ds(base, BPW)])


def kernel(x, W, H):
    return _mf_dot_kernel(x[:, 0], x[:, 1], W.T, H.T)


# speedup vs baseline: 1.0009x; 1.0002x over previous
import functools

import jax
import jax.numpy as jnp
from jax import lax
from jax.experimental import pallas as pl
from jax.experimental.pallas import tpu as pltpu
from jax.experimental.pallas import tpu_sc as plsc

BATCH = 16384
EMBED = 32
NUM_CORES = 2
NUM_SUBCORES = 16
LANES = 16
NUM_WORKERS = NUM_CORES * NUM_SUBCORES
BPW = BATCH // NUM_WORKERS
NBUF = 4
AHEAD = NBUF - 1

_mesh = plsc.VectorSubcoreMesh(core_axis_name="c", subcore_axis_name="s")


@functools.partial(
    pl.kernel,
    out_type=jax.ShapeDtypeStruct((BATCH,), jnp.float32),
    mesh=_mesh,
    compiler_params=pltpu.CompilerParams(needs_layout_passes=False,
                                         use_tc_tiling_on_sc=True),
    scratch_types=[
        pltpu.VMEM((BPW + LANES,), jnp.int32),
        pltpu.VMEM((BPW + LANES,), jnp.int32),
        pltpu.VMEM((NBUF, EMBED, 128), jnp.float32),
        pltpu.VMEM((NBUF, EMBED, 128), jnp.float32),
        pltpu.VMEM((BPW,), jnp.float32),
    ] + [pltpu.SemaphoreType.DMA] * (2 * NBUF),
)
def _mf_dot_kernel(xu_hbm, xi_hbm, wt_hbm, ht_hbm, out_hbm,
                   uidx, vidx, wtile, htile, outv, *sems):
    semw = sems[:NBUF]
    semh = sems[NBUF:]
    wid = lax.axis_index("s") * NUM_CORES + lax.axis_index("c")
    base = wid * BPW
    lanes = lax.iota(jnp.int32, LANES)

    pltpu.sync_copy(xu_hbm.at[pl.ds(base, BPW)], uidx.at[pl.ds(0, BPW)])
    pltpu.sync_copy(xi_hbm.at[pl.ds(base, BPW)], vidx.at[pl.ds(0, BPW)])
    uidx[pl.ds(BPW, LANES)] = jnp.zeros((LANES,), jnp.int32)
    vidx[pl.ds(BPW, LANES)] = jnp.zeros((LANES,), jnp.int32)

    def issue(u, v, slot):
        cu = pl.multiple_of((u >> 7) << 7, 128)
        cv = pl.multiple_of((v >> 7) << 7, 128)
        pltpu.async_copy(wt_hbm.at[:, pl.ds(cu, 128)], wtile.at[slot],
                         semw[slot])
        pltpu.async_copy(ht_hbm.at[:, pl.ds(cv, 128)], htile.at[slot],
                         semh[slot])

    def drain(slot):
        pltpu.make_async_copy(wt_hbm.at[:, pl.ds(0, 128)],
                              wtile.at[slot], semw[slot]).wait()
        pltpu.make_async_copy(ht_hbm.at[:, pl.ds(0, 128)],
                              htile.at[slot], semh[slot]).wait()

    uvec0 = uidx[pl.ds(0, LANES)]
    vvec0 = vidx[pl.ds(0, LANES)]
    for p in range(AHEAD):
        issue(uvec0[p], vvec0[p], p)

    def group_body(g, carry):
        uvec, vvec = carry
        acc = jnp.zeros((LANES,), jnp.float32)
        unext, vnext = uvec, vvec
        for j in range(LANES):
            if j == LANES - AHEAD:
                unext = uidx[pl.ds((g + 1) * LANES, LANES)]
                vnext = vidx[pl.ds((g + 1) * LANES, LANES)]
            p = j + AHEAD
            if p < LANES:
                nu, nv = uvec[p], vvec[p]
            else:
                nu, nv = unext[p - LANES], vnext[p - LANES]
            slot = j % NBUF
            issue(nu, nv, p % NBUF)
            drain(slot)
            lu = jnp.broadcast_to(uvec[j] & 127, (LANES,))
            lv = jnp.broadcast_to(vvec[j] & 127, (LANES,))
            w0 = plsc.load_gather(wtile.at[slot], [lanes, lu])
            w1 = plsc.load_gather(wtile.at[slot], [lanes + LANES, lu])
            h0 = plsc.load_gather(htile.at[slot], [lanes, lv])
            h1 = plsc.load_gather(htile.at[slot], [lanes + LANES, lv])
            s = jnp.sum(w0 * h0 + w1 * h1)
            acc = jnp.where(lanes == j, s, acc)
        outv[pl.ds(g * LANES, LANES)] = acc
        return unext, vnext

    lax.fori_loop(0, BPW // LANES, group_body, (uvec0, vvec0))
    for p in range(AHEAD):
        drain((BPW + p) % NBUF)

    pltpu.sync_copy(outv, out_hbm.at[pl.ds(base, BPW)])


def kernel(x, W, H):
    return _mf_dot_kernel(x[:, 0], x[:, 1], W.T, H.T)


# speedup vs baseline: 1.0022x; 1.0013x over previous
import functools

import jax
import jax.numpy as jnp
from jax import lax
from jax.experimental import pallas as pl
from jax.experimental.pallas import tpu as pltpu
from jax.experimental.pallas import tpu_sc as plsc

BATCH = 16384
EMBED = 32
NUM_CORES = 2
NUM_SUBCORES = 16
LANES = 16
NUM_WORKERS = NUM_CORES * NUM_SUBCORES
BPW = BATCH // NUM_WORKERS
NBUF = 4
AHEAD = NBUF - 1

_mesh = plsc.VectorSubcoreMesh(core_axis_name="c", subcore_axis_name="s")


@functools.partial(
    pl.kernel,
    out_type=jax.ShapeDtypeStruct((BATCH,), jnp.float32),
    mesh=_mesh,
    compiler_params=pltpu.CompilerParams(needs_layout_passes=False,
                                         use_tc_tiling_on_sc=True),
    scratch_types=[
        pltpu.VMEM((BPW + LANES,), jnp.int32),
        pltpu.VMEM((BPW + LANES,), jnp.int32),
        pltpu.VMEM((NBUF, EMBED, 128), jnp.float32),
        pltpu.VMEM((NBUF, EMBED, 128), jnp.float32),
        pltpu.VMEM((BPW,), jnp.float32),
    ] + [pltpu.SemaphoreType.DMA] * (2 * NBUF),
)
def _mf_dot_kernel(xt_hbm, wt_hbm, ht_hbm, out_hbm,
                   uidx, vidx, wtile, htile, outv, *sems):
    semw = sems[:NBUF]
    semh = sems[NBUF:]
    wid = lax.axis_index("s") * NUM_CORES + lax.axis_index("c")
    base = wid * BPW
    lanes = lax.iota(jnp.int32, LANES)

    xbase = pl.multiple_of(base, 128)
    pltpu.sync_copy(xt_hbm.at[0, pl.ds(xbase, BPW)], uidx.at[pl.ds(0, BPW)])
    pltpu.sync_copy(xt_hbm.at[1, pl.ds(xbase, BPW)], vidx.at[pl.ds(0, BPW)])
    uidx[pl.ds(BPW, LANES)] = jnp.zeros((LANES,), jnp.int32)
    vidx[pl.ds(BPW, LANES)] = jnp.zeros((LANES,), jnp.int32)

    def issue(u, v, slot):
        cu = pl.multiple_of((u >> 7) << 7, 128)
        cv = pl.multiple_of((v >> 7) << 7, 128)
        pltpu.async_copy(wt_hbm.at[:, pl.ds(cu, 128)], wtile.at[slot],
                         semw[slot])
        pltpu.async_copy(ht_hbm.at[:, pl.ds(cv, 128)], htile.at[slot],
                         semh[slot])

    def drain(slot):
        pltpu.make_async_copy(wt_hbm.at[:, pl.ds(0, 128)],
                              wtile.at[slot], semw[slot]).wait()
        pltpu.make_async_copy(ht_hbm.at[:, pl.ds(0, 128)],
                              htile.at[slot], semh[slot]).wait()

    uvec0 = uidx[pl.ds(0, LANES)]
    vvec0 = vidx[pl.ds(0, LANES)]
    for p in range(AHEAD):
        issue(uvec0[p], vvec0[p], p)

    def group_body(g, carry):
        uvec, vvec = carry
        acc = jnp.zeros((LANES,), jnp.float32)
        unext, vnext = uvec, vvec
        for j in range(LANES):
            if j == LANES - AHEAD:
                unext = uidx[pl.ds((g + 1) * LANES, LANES)]
                vnext = vidx[pl.ds((g + 1) * LANES, LANES)]
            p = j + AHEAD
            if p < LANES:
                nu, nv = uvec[p], vvec[p]
            else:
                nu, nv = unext[p - LANES], vnext[p - LANES]
            slot = j % NBUF
            issue(nu, nv, p % NBUF)
            drain(slot)
            lu = jnp.broadcast_to(uvec[j] & 127, (LANES,))
            lv = jnp.broadcast_to(vvec[j] & 127, (LANES,))
            w0 = plsc.load_gather(wtile.at[slot], [lanes, lu])
            w1 = plsc.load_gather(wtile.at[slot], [lanes + LANES, lu])
            h0 = plsc.load_gather(htile.at[slot], [lanes, lv])
            h1 = plsc.load_gather(htile.at[slot], [lanes + LANES, lv])
            s = jnp.sum(w0 * h0 + w1 * h1)
            acc = jnp.where(lanes == j, s, acc)
        outv[pl.ds(g * LANES, LANES)] = acc
        return unext, vnext

    lax.fori_loop(0, BPW // LANES, group_body, (uvec0, vvec0))
    for p in range(AHEAD):
        drain((BPW + p) % NBUF)

    pltpu.sync_copy(outv, out_hbm.at[pl.ds(base, BPW)])


def kernel(x, W, H):
    return _mf_dot_kernel(x.T, W.T, H.T)
